# baseline (speedup 1.0000x reference)
import math

import numpy as np
import jax
import jax.numpy as jnp
from jax.experimental import pallas as pl
from jax.experimental.pallas import tpu as pltpu

F32 = jnp.float32
BF16 = jnp.bfloat16

B = 16
D = 768
NH = 12
DH = 64
GRAN = 8
NE = 8
NCLS = 7
LN_EPS = 1e-5


PREC = dict(conv=2, qkvo=1, attn=1, ffn=1, moe=2, fusion=3, head=3)


def _dot(a, b, dims):
    return jax.lax.dot_general(a, b, (dims, ((), ())),
                               preferred_element_type=F32)


def _split(a):
    ah = a.astype(BF16)
    return ah, (a - ah.astype(F32)).astype(BF16)


def _mm(a, b, p=1):
    dims = ((1,), (0,))
    if p == 3:
        ah, al = _split(a)
        bh, bl = _split(b)
        return (_dot(ah, bh, dims) + _dot(al, bh, dims)
                + _dot(ah, bl, dims))
    if p == 2:
        ah, al = _split(a)
        m = a.shape[0]
        y = _dot(jnp.concatenate([ah, al], axis=0), b.astype(BF16), dims)
        return y[0:m] + y[m:2 * m]
    if p == 4:
        ah = a.astype(BF16)
        bh, bl = _split(b)
        return _dot(ah, bh, dims) + _dot(ah, bl, dims)
    return _dot(a.astype(BF16), b.astype(BF16), dims)


def _mm_t(a, b, p=1):
    dims = ((1,), (1,))
    if p == 3:
        ah, al = _split(a)
        bh, bl = _split(b)
        return (_dot(ah, bh, dims) + _dot(al, bh, dims)
                + _dot(ah, bl, dims))
    return _dot(a.astype(BF16), b.astype(BF16), dims)


def _ln(x, g, b):
    m = jnp.mean(x, axis=-1, keepdims=True)
    v = jnp.mean((x - m) ** 2, axis=-1, keepdims=True)
    return (x - m) * jax.lax.rsqrt(v + LN_EPS) * g + b


def _sinusoid_np(S, d):
    pos = np.arange(S)[:, None].astype(np.float64)
    i = np.arange(d)[None, :]
    angle = pos / np.power(10000.0, (2 * (i // 2)) / d)
    pe = np.where(i % 2 == 0, np.sin(angle), np.cos(angle))
    return jnp.asarray(pe, F32)


def _conv_pe_kernel(xp_ref, w_ref, pe_ref, o_ref, *, S):
    x = xp_ref[0]
    p = PREC['conv']
    y = (_mm(x[0:S], w_ref[0], p) + _mm(x[1:S + 1], w_ref[1], p)
         + _mm(x[2:S + 2], w_ref[2], p))
    o_ref[0] = y * np.float32(math.sqrt(D)) + pe_ref[...]


def _conv_pe(x, conv_w, S, Cin):
    xp = jnp.pad(x, ((0, 0), (1, 1), (0, 0)))
    wt = jnp.transpose(conv_w, (2, 1, 0)).astype(BF16)
    pe = _sinusoid_np(S, D)
    return pl.pallas_call(
        lambda a, b, c, o: _conv_pe_kernel(a, b, c, o, S=S),
        grid=(B,),
        in_specs=[
            pl.BlockSpec((1, S + 2, Cin), lambda b: (b, 0, 0)),
            pl.BlockSpec((3, Cin, D), lambda b: (0, 0, 0)),
            pl.BlockSpec((S, D), lambda b: (0, 0)),
        ],
        out_specs=pl.BlockSpec((1, S, D), lambda b: (b, 0, 0)),
        out_shape=jax.ShapeDtypeStruct((B, S, D), F32),
    )(xp, wt, pe)


def _attn_kernel(x_ref, wq_ref, wk_ref, wv_ref, wo_ref,
                 bq_ref, bk_ref, bv_ref, bo_ref, g_ref, b_ref, o_ref):
    x = x_ref[0]
    pq, pa = PREC['qkvo'], PREC['attn']
    h = _ln(x, g_ref[...], b_ref[...])
    q = _mm(h, wq_ref[...], pq) + bq_ref[...]
    k = _mm(h, wk_ref[...], pq) + bk_ref[...]
    v = _mm(h, wv_ref[...], pq) + bv_ref[...]
    outs = []
    for hh in range(NH):
        sl = slice(hh * DH, (hh + 1) * DH)
        s = _mm_t(q[:, sl], k[:, sl], pa) * np.float32(DH ** -0.5)
        p = jax.nn.softmax(s, axis=-1)
        outs.append(_mm(p, v[:, sl], pa))
    att = jnp.concatenate(outs, axis=1)
    o_ref[0] = x + _mm(att, wo_ref[...], pq) + bo_ref[...]


def _attn_preln(x, lp, S):
    args = (x, lp['q']['w'].astype(BF16), lp['k']['w'].astype(BF16),
            lp['v']['w'].astype(BF16), lp['o']['w'].astype(BF16),
            lp['q']['b'].reshape(1, D), lp['k']['b'].reshape(1, D),
            lp['v']['b'].reshape(1, D), lp['o']['b'].reshape(1, D),
            lp['ln1g'].reshape(1, D), lp['ln1b'].reshape(1, D))
    w_spec = pl.BlockSpec((D, D), lambda b: (0, 0))
    v_spec = pl.BlockSpec((1, D), lambda b: (0, 0))
    return pl.pallas_call(
        _attn_kernel,
        grid=(B,),
        in_specs=[pl.BlockSpec((1, S, D), lambda b: (b, 0, 0)),
                  w_spec, w_spec, w_spec, w_spec,
                  v_spec, v_spec, v_spec, v_spec, v_spec, v_spec],
        out_specs=pl.BlockSpec((1, S, D), lambda b: (b, 0, 0)),
        out_shape=jax.ShapeDtypeStruct((B, S, D), F32),
    )(*args)


def _ffn_kernel(x_ref, w1_ref, b1_ref, w2_ref, b2_ref, g_ref, b_ref, o_ref):
    x = x_ref[...]
    p = PREC['ffn']
    h = _ln(x, g_ref[...], b_ref[...])
    u = jnp.maximum(_mm(h, w1_ref[...], p) + b1_ref[...], 0.0)
    o_ref[...] = x + _mm(u, w2_ref[...], p) + b2_ref[...]


def _ffn_preln(x, lp, S, hidden, tile):
    n = B * S
    xf = x.reshape(n, D)
    nt = n // tile
    c_spec = lambda shape: pl.BlockSpec(shape, lambda i: (0, 0))
    out = pl.pallas_call(
        _ffn_kernel,
        grid=(nt,),
        in_specs=[pl.BlockSpec((tile, D), lambda i: (i, 0)),
                  c_spec((D, hidden)), c_spec((1, hidden)),
                  c_spec((hidden, D)), c_spec((1, D)),
                  c_spec((1, D)), c_spec((1, D))],
        out_specs=pl.BlockSpec((tile, D), lambda i: (i, 0)),
        out_shape=jax.ShapeDtypeStruct((n, D), F32),
    )(xf, lp['ff1']['w'].astype(BF16), lp['ff1']['b'].reshape(1, hidden),
      lp['ff2']['w'].astype(BF16), lp['ff2']['b'].reshape(1, D),
      lp['ln2g'].reshape(1, D), lp['ln2b'].reshape(1, D))
    return out.reshape(B, S, D)


def _lnpool_kernel(x_ref, g_ref, b_ref, co_ref, fi_ref, *, S):
    x = _ln(x_ref[0], g_ref[...], b_ref[...])
    co_ref[0] = jnp.max(x, axis=0, keepdims=True)
    bs = S // GRAN
    for g in range(GRAN):
        fi_ref[0, g:g + 1, :] = jnp.max(
            x[g * bs:(g + 1) * bs], axis=0, keepdims=True)


def _lnpool(x, g, b, S):
    co, fi = pl.pallas_call(
        lambda a, c, d, e, f: _lnpool_kernel(a, c, d, e, f, S=S),
        grid=(B,),
        in_specs=[pl.BlockSpec((1, S, D), lambda i: (i, 0, 0)),
                  pl.BlockSpec((1, D), lambda i: (0, 0)),
                  pl.BlockSpec((1, D), lambda i: (0, 0))],
        out_specs=[pl.BlockSpec((1, 1, D), lambda i: (i, 0, 0)),
                   pl.BlockSpec((1, GRAN, D), lambda i: (i, 0, 0))],
        out_shape=[jax.ShapeDtypeStruct((B, 1, D), F32),
                   jax.ShapeDtypeStruct((B, GRAN, D), F32)],
    )(x, g.reshape(1, D), b.reshape(1, D))
    return co.reshape(B, D), fi


def _moe_kernel(x_ref, wg_ref, b1_ref, b2_ref, *rest):
    w1s = rest[0:NE]
    w2s = rest[NE:2 * NE]
    o_ref, loss_ref, w1buf, w2buf, sem1, sem2 = rest[2 * NE:]

    def start(e, slot):
        pltpu.make_async_copy(w1s[e], w1buf.at[slot], sem1.at[slot]).start()
        pltpu.make_async_copy(w2s[e], w2buf.at[slot], sem2.at[slot]).start()

    def wait(e, slot):
        pltpu.make_async_copy(w1s[e], w1buf.at[slot], sem1.at[slot]).wait()
        pltpu.make_async_copy(w2s[e], w2buf.at[slot], sem2.at[slot]).wait()

    start(0, 0)

    x = x_ref[...]
    logits = _mm(x, wg_ref[...], 1)
    iota = jax.lax.broadcasted_iota(jnp.int32, (B, NE), 1)
    m1 = jnp.max(logits, axis=1, keepdims=True)
    idx1 = jnp.min(jnp.where(logits == m1, iota, NE + 1), axis=1,
                   keepdims=True)
    sel1 = iota == idx1
    masked = jnp.where(sel1, -jnp.inf, logits)
    m2 = jnp.max(masked, axis=1, keepdims=True)
    idx2 = jnp.min(jnp.where(masked == m2, iota, NE + 1), axis=1,
                   keepdims=True)
    sel2 = iota == idx2
    ex = jnp.exp(m2 - m1)
    denom = 1.0 + ex
    gates = (jnp.where(sel1, 1.0 / denom, 0.0)
             + jnp.where(sel2, ex / denom, 0.0))

    def cv_sq(v):
        m = jnp.mean(v)
        var = jnp.sum((v - m) ** 2) / np.float32(NE - 1)
        return var / (m * m + np.float32(1e-10))

    importance = jnp.sum(gates, axis=0)
    load = jnp.sum((gates > 0.0).astype(F32), axis=0)
    loss_ref[...] = jnp.full((1, 1),
                             (cv_sq(importance) + cv_sq(load)) * 0.01, F32)

    pm = PREC['moe']
    acc = jnp.zeros((B, D), F32)
    for e in range(NE):
        if e + 1 < NE:
            if e == 0:
                start(1, 1)
            else:
                guard = jnp.where(acc[0, 0] == np.float32(np.inf),
                                  1, 0).astype(jnp.int32)
                slot = (e + 1) % 2 + guard
                pltpu.make_async_copy(
                    w1s[e + 1], w1buf.at[slot],
                    sem1.at[(e + 1) % 2]).start()
                pltpu.make_async_copy(
                    w2s[e + 1], w2buf.at[slot],
                    sem2.at[(e + 1) % 2]).start()
        wait(e, e % 2)
        h = jnp.maximum(_mm(x, w1buf[e % 2], pm) + b1_ref[e:e + 1], 0.0)
        y = _mm(h, w2buf[e % 2], pm) + b2_ref[e:e + 1]
        acc = acc + gates[:, e:e + 1] * y
    o_ref[...] = acc


def _moe_call(x, p):
    in_size = D * GRAN
    b1 = jnp.stack([ep['fc1']['b'] for ep in p['experts']])
    b2 = jnp.stack([ep['fc2']['b'] for ep in p['experts']])
    any_spec = pl.BlockSpec(memory_space=pl.ANY)
    vmem_spec = pl.BlockSpec(memory_space=pltpu.MemorySpace.VMEM)
    y, loss = pl.pallas_call(
        _moe_kernel,
        in_specs=[vmem_spec, vmem_spec, vmem_spec, vmem_spec]
                 + [any_spec] * (2 * NE),
        out_shape=[jax.ShapeDtypeStruct((B, D), F32),
                   jax.ShapeDtypeStruct((1, 1), F32)],
        scratch_shapes=[pltpu.VMEM((2, in_size, D), BF16),
                        pltpu.VMEM((2, D, D), BF16),
                        pltpu.SemaphoreType.DMA((2,)),
                        pltpu.SemaphoreType.DMA((2,))],
    )(x, p['w_gate'], b1, b2,
      *[ep['fc1']['w'].astype(BF16) for ep in p['experts']],
      *[ep['fc2']['w'].astype(BF16) for ep in p['experts']])
    return y, loss


def _fusion_kernel(x_ref, wq_ref, wk_ref, wv_ref, wo_ref,
                   bq_ref, bk_ref, bv_ref, bo_ref,
                   g1_ref, b1_ref, w1_ref, f1_ref, w2_ref, f2_ref,
                   g2_ref, b2_ref, o_ref, *, SEQ):
    n = B * SEQ
    p = PREC['fusion']
    x = x_ref[...]
    q = _mm(x, wq_ref[...], p) + bq_ref[...]
    k = _mm(x, wk_ref[...], p) + bk_ref[...]
    v = _mm(x, wv_ref[...], p) + bv_ref[...]
    bi = jax.lax.broadcasted_iota(jnp.int32, (n, n), 0) // SEQ
    bj = jax.lax.broadcasted_iota(jnp.int32, (n, n), 1) // SEQ
    same = bi == bj
    outs = []
    for hh in range(NH):
        sl = slice(hh * DH, (hh + 1) * DH)
        s = _mm_t(q[:, sl], k[:, sl], p) * np.float32(DH ** -0.5)
        s = jnp.where(same, s, -jnp.inf)
        pr = jax.nn.softmax(s, axis=-1)
        outs.append(_mm(pr, v[:, sl], p))
    att = jnp.concatenate(outs, axis=1)
    x = _ln(x + _mm(att, wo_ref[...], p) + bo_ref[...], g1_ref[...], b1_ref[...])
    u = jnp.maximum(_mm(x, w1_ref[...], p) + f1_ref[...], 0.0)
    x = _ln(x + _mm(u, w2_ref[...], p) + f2_ref[...], g2_ref[...], b2_ref[...])
    o_ref[...] = x


def _fusion_layer(x, lp, SEQ, hidden):
    return pl.pallas_call(
        lambda *a: _fusion_kernel(*a, SEQ=SEQ),
        out_shape=jax.ShapeDtypeStruct((B * SEQ, D), F32),
    )(x, lp['q']['w'], lp['k']['w'], lp['v']['w'], lp['o']['w'],
      lp['q']['b'].reshape(1, D), lp['k']['b'].reshape(1, D),
      lp['v']['b'].reshape(1, D), lp['o']['b'].reshape(1, D),
      lp['ln1g'].reshape(1, D), lp['ln1b'].reshape(1, D),
      lp['ff1']['w'], lp['ff1']['b'].reshape(1, hidden),
      lp['ff2']['w'], lp['ff2']['b'].reshape(1, D),
      lp['ln2g'].reshape(1, D), lp['ln2b'].reshape(1, D))


def _head_kernel(h0_ref, h1_ref, h2_ref, h3_ref, h4_ref, h5_ref,
                 mcs_w_ref, mcs_b_ref, fcs_w_ref, fcs_b_ref,
                 mfs_w_ref, mfs_b_ref, ffs_w_ref, ffs_b_ref,
                 all_w_ref, all_b_ref, ts_w_ref, ts_b_ref,
                 as_w_ref, as_b_ref, vs_w_ref, vs_b_ref,
                 cls_w_ref, cls_b_ref, o_ref):
    h0, h1, h2 = h0_ref[...], h1_ref[...], h2_ref[...]
    h3, h4, h5 = h3_ref[...], h4_ref[...], h5_ref[...]
    scale = np.float32(D ** -0.5)

    ph = PREC['head']

    def attn_mod(f1, f2, f3, mw, mb, fw, fb):
        a = jnp.maximum(
            _mm(jnp.concatenate([f1, f2, f3], axis=1), mw, ph) + mb, 0.0)
        s = (_mm(a, fw, ph) + fb) * scale
        w = jax.nn.softmax(s, axis=1)
        fused = (f1 * w[:, 0:1] + f2 * w[:, 1:2] + f3 * w[:, 2:3])
        return w, fused

    aw_cs, a_cs = attn_mod(h0, h2, h4, mcs_w_ref[...], mcs_b_ref[...],
                           fcs_w_ref[...], fcs_b_ref[...])
    _, a_fs = attn_mod(h1, h3, h5, mfs_w_ref[...], mfs_b_ref[...],
                       ffs_w_ref[...], ffs_b_ref[...])
    fea_cfs = _mm(jnp.concatenate([a_cs, a_fs], axis=1),
                  all_w_ref[...], ph) + all_b_ref[...]
    fm_t = _mm(jnp.concatenate([h0, h1], axis=1), ts_w_ref[...], ph) + ts_b_ref[...]
    fm_a = _mm(jnp.concatenate([h2, h3], axis=1), as_w_ref[...], ph) + as_b_ref[...]
    fm_v = _mm(jnp.concatenate([h4, h5], axis=1), vs_w_ref[...], ph) + vs_b_ref[...]
    fcf = (fm_t * aw_cs[:, 0:1] + fm_a * aw_cs[:, 1:2] + fm_v * aw_cs[:, 2:3])
    o_ref[...] = _mm(jnp.concatenate([fea_cfs, fcf], axis=1),
                     cls_w_ref[...], ph) + cls_b_ref[...]


def _head(hs, params):
    p = params
    return pl.pallas_call(
        _head_kernel,
        out_shape=jax.ShapeDtypeStruct((B, NCLS), F32),
    )(*hs,
      p['attn_cs']['mlp']['w'], p['attn_cs']['mlp']['b'].reshape(1, D),
      p['attn_cs']['fc']['w'], p['attn_cs']['fc']['b'].reshape(1, 3),
      p['attn_fs']['mlp']['w'], p['attn_fs']['mlp']['b'].reshape(1, D),
      p['attn_fs']['fc']['w'], p['attn_fs']['fc']['b'].reshape(1, 3),
      p['fc_all']['w'], p['fc_all']['b'].reshape(1, D),
      p['fc_ts']['w'], p['fc_ts']['b'].reshape(1, D),
      p['fc_as']['w'], p['fc_as']['b'].reshape(1, D),
      p['fc_vs']['w'], p['fc_vs']['b'].reshape(1, D),
      p['classifier']['w'], p['classifier']['b'].reshape(1, NCLS))


def _encoder(x, p, S, Cin, tile):
    h = _conv_pe(x, p['conv'], S, Cin)
    for lp in p['enc']['layers']:
        h = _attn_preln(h, lp, S)
        h = _ffn_preln(h, lp, S, 4 * D, tile)
    co, fi = _lnpool(h, p['enc']['lnfg'], p['enc']['lnfb'], S)
    return co, fi


def _moe(fine, p):
    x = fine.transpose(0, 2, 1).reshape(B, D * GRAN)
    return _moe_call(x, p)


def kernel(x_t, x_a, x_v, label_t, label_a, label_v, label_m, params):
    x_t = x_t[:, 0:80, :].astype(F32)
    x_a = x_a.astype(F32)
    x_v = x_v.astype(F32)

    co_a, fi_a = _encoder(x_a, params['audio'], 256, 1024, 256)
    co_v, fi_v = _encoder(x_v, params['video'], 256, 2048, 256)
    co_t, fi_t = _encoder(x_t, params['text'], 80, 1024, 256)

    fine_ts, l_ts = _moe(fi_t, params['moe_t'])
    fine_as, l_as = _moe(fi_a, params['moe_a'])
    fine_vs, l_vs = _moe(fi_v, params['moe_v'])

    h = jnp.stack([co_t, fine_ts, co_a, fine_as, co_v, fine_vs],
                  axis=1).reshape(B * 6, D)
    for lp in params['trans']:
        h = _fusion_layer(h, lp, 6, 2048)
    h = h.reshape(B, 6, D)
    hs = [h[:, i, :] for i in range(6)]

    logits = _head(hs, params)
    moes = (l_ts + l_as + l_vs).reshape(())
    return (moes, logits)


# speedup vs baseline: 1.1725x; 1.1725x over previous
import math

import numpy as np
import jax
import jax.numpy as jnp
from jax.experimental import pallas as pl
from jax.experimental.pallas import tpu as pltpu

F32 = jnp.float32
BF16 = jnp.bfloat16

B = 16
D = 768
NH = 12
DH = 64
GRAN = 8
NE = 8
NCLS = 7
LN_EPS = 1e-5


PREC = dict(conv=2, qkvo=1, attn=1, ffn=1, moe=2, fusion=3, head=3)


def _dot(a, b, dims):
    return jax.lax.dot_general(a, b, (dims, ((), ())),
                               preferred_element_type=F32)


def _split(a):
    ah = a.astype(BF16)
    return ah, (a - ah.astype(F32)).astype(BF16)


def _mm(a, b, p=1):
    dims = ((1,), (0,))
    if p == 3:
        ah, al = _split(a)
        bh, bl = _split(b)
        return (_dot(ah, bh, dims) + _dot(al, bh, dims)
                + _dot(ah, bl, dims))
    if p == 2:
        ah, al = _split(a)
        m = a.shape[0]
        y = _dot(jnp.concatenate([ah, al], axis=0), b.astype(BF16), dims)
        return y[0:m] + y[m:2 * m]
    if p == 4:
        ah = a.astype(BF16)
        bh, bl = _split(b)
        return _dot(ah, bh, dims) + _dot(ah, bl, dims)
    return _dot(a.astype(BF16), b.astype(BF16), dims)


def _mm_t(a, b, p=1):
    dims = ((1,), (1,))
    if p == 3:
        ah, al = _split(a)
        bh, bl = _split(b)
        return (_dot(ah, bh, dims) + _dot(al, bh, dims)
                + _dot(ah, bl, dims))
    return _dot(a.astype(BF16), b.astype(BF16), dims)


def _ln(x, g, b):
    m = jnp.mean(x, axis=-1, keepdims=True)
    v = jnp.mean((x - m) ** 2, axis=-1, keepdims=True)
    return (x - m) * jax.lax.rsqrt(v + LN_EPS) * g + b


def _sinusoid_np(S, d):
    pos = np.arange(S)[:, None].astype(np.float64)
    i = np.arange(d)[None, :]
    angle = pos / np.power(10000.0, (2 * (i // 2)) / d)
    pe = np.where(i % 2 == 0, np.sin(angle), np.cos(angle))
    return jnp.asarray(pe, F32)


def _conv_pe_kernel(xp_ref, w_ref, pe_ref, o_ref, *, S):
    x = xp_ref[0]
    p = PREC['conv']
    y = (_mm(x[0:S], w_ref[0], p) + _mm(x[1:S + 1], w_ref[1], p)
         + _mm(x[2:S + 2], w_ref[2], p))
    o_ref[0] = y * np.float32(math.sqrt(D)) + pe_ref[...]


def _conv_pe(x, conv_w, S, Cin):
    xp = jnp.pad(x, ((0, 0), (1, 1), (0, 0)))
    wt = jnp.transpose(conv_w, (2, 1, 0))
    pe = _sinusoid_np(S, D)
    return pl.pallas_call(
        lambda a, b, c, o: _conv_pe_kernel(a, b, c, o, S=S),
        grid=(B,),
        in_specs=[
            pl.BlockSpec((1, S + 2, Cin), lambda b: (b, 0, 0)),
            pl.BlockSpec((3, Cin, D), lambda b: (0, 0, 0)),
            pl.BlockSpec((S, D), lambda b: (0, 0)),
        ],
        out_specs=pl.BlockSpec((1, S, D), lambda b: (b, 0, 0)),
        out_shape=jax.ShapeDtypeStruct((B, S, D), F32),
    )(xp, wt, pe)


def _attn_kernel(x_ref, wq_ref, wk_ref, wv_ref, wo_ref,
                 bq_ref, bk_ref, bv_ref, bo_ref, g_ref, b_ref, o_ref):
    x = x_ref[0]
    pq, pa = PREC['qkvo'], PREC['attn']
    h = _ln(x, g_ref[...], b_ref[...])
    q = _mm(h, wq_ref[...], pq) + bq_ref[...]
    k = _mm(h, wk_ref[...], pq) + bk_ref[...]
    v = _mm(h, wv_ref[...], pq) + bv_ref[...]
    outs = []
    for hh in range(NH):
        sl = slice(hh * DH, (hh + 1) * DH)
        s = _mm_t(q[:, sl], k[:, sl], pa) * np.float32(DH ** -0.5)
        p = jax.nn.softmax(s, axis=-1)
        outs.append(_mm(p, v[:, sl], pa))
    att = jnp.concatenate(outs, axis=1)
    o_ref[0] = x + _mm(att, wo_ref[...], pq) + bo_ref[...]


def _attn_preln(x, lp, S):
    args = (x, lp['q']['w'], lp['k']['w'], lp['v']['w'], lp['o']['w'],
            lp['q']['b'].reshape(1, D), lp['k']['b'].reshape(1, D),
            lp['v']['b'].reshape(1, D), lp['o']['b'].reshape(1, D),
            lp['ln1g'].reshape(1, D), lp['ln1b'].reshape(1, D))
    w_spec = pl.BlockSpec((D, D), lambda b: (0, 0))
    v_spec = pl.BlockSpec((1, D), lambda b: (0, 0))
    return pl.pallas_call(
        _attn_kernel,
        grid=(B,),
        in_specs=[pl.BlockSpec((1, S, D), lambda b: (b, 0, 0)),
                  w_spec, w_spec, w_spec, w_spec,
                  v_spec, v_spec, v_spec, v_spec, v_spec, v_spec],
        out_specs=pl.BlockSpec((1, S, D), lambda b: (b, 0, 0)),
        out_shape=jax.ShapeDtypeStruct((B, S, D), F32),
    )(*args)


def _ffn_kernel(x_ref, w1_ref, b1_ref, w2_ref, b2_ref, g_ref, b_ref, o_ref):
    x = x_ref[...]
    p = PREC['ffn']
    h = _ln(x, g_ref[...], b_ref[...])
    u = jnp.maximum(_mm(h, w1_ref[...], p) + b1_ref[...], 0.0)
    o_ref[...] = x + _mm(u, w2_ref[...], p) + b2_ref[...]


def _ffn_preln(x, lp, S, hidden, tile):
    n = B * S
    xf = x.reshape(n, D)
    nt = n // tile
    c_spec = lambda shape: pl.BlockSpec(shape, lambda i: (0, 0))
    out = pl.pallas_call(
        _ffn_kernel,
        grid=(nt,),
        in_specs=[pl.BlockSpec((tile, D), lambda i: (i, 0)),
                  c_spec((D, hidden)), c_spec((1, hidden)),
                  c_spec((hidden, D)), c_spec((1, D)),
                  c_spec((1, D)), c_spec((1, D))],
        out_specs=pl.BlockSpec((tile, D), lambda i: (i, 0)),
        out_shape=jax.ShapeDtypeStruct((n, D), F32),
    )(xf, lp['ff1']['w'], lp['ff1']['b'].reshape(1, hidden),
      lp['ff2']['w'], lp['ff2']['b'].reshape(1, D),
      lp['ln2g'].reshape(1, D), lp['ln2b'].reshape(1, D))
    return out.reshape(B, S, D)


def _lnpool_kernel(x_ref, g_ref, b_ref, co_ref, fi_ref, *, S):
    x = _ln(x_ref[0], g_ref[...], b_ref[...])
    co_ref[0] = jnp.max(x, axis=0, keepdims=True)
    bs = S // GRAN
    for g in range(GRAN):
        fi_ref[0, g:g + 1, :] = jnp.max(
            x[g * bs:(g + 1) * bs], axis=0, keepdims=True)


def _lnpool(x, g, b, S):
    co, fi = pl.pallas_call(
        lambda a, c, d, e, f: _lnpool_kernel(a, c, d, e, f, S=S),
        grid=(B,),
        in_specs=[pl.BlockSpec((1, S, D), lambda i: (i, 0, 0)),
                  pl.BlockSpec((1, D), lambda i: (0, 0)),
                  pl.BlockSpec((1, D), lambda i: (0, 0))],
        out_specs=[pl.BlockSpec((1, 1, D), lambda i: (i, 0, 0)),
                   pl.BlockSpec((1, GRAN, D), lambda i: (i, 0, 0))],
        out_shape=[jax.ShapeDtypeStruct((B, 1, D), F32),
                   jax.ShapeDtypeStruct((B, GRAN, D), F32)],
    )(x, g.reshape(1, D), b.reshape(1, D))
    return co.reshape(B, D), fi


def _moe_kernel(x_ref, wg_ref, b1_ref, b2_ref, *rest):
    w1s = rest[0:NE]
    w2s = rest[NE:2 * NE]
    o_ref, loss_ref, w1buf, w2buf, sem1, sem2 = rest[2 * NE:]

    def start(e, slot):
        pltpu.make_async_copy(w1s[e], w1buf.at[slot], sem1.at[slot]).start()
        pltpu.make_async_copy(w2s[e], w2buf.at[slot], sem2.at[slot]).start()

    def wait(e, slot):
        pltpu.make_async_copy(w1s[e], w1buf.at[slot], sem1.at[slot]).wait()
        pltpu.make_async_copy(w2s[e], w2buf.at[slot], sem2.at[slot]).wait()

    start(0, 0)

    x = x_ref[...]
    logits = _mm(x, wg_ref[...], 1)
    iota = jax.lax.broadcasted_iota(jnp.int32, (B, NE), 1)
    m1 = jnp.max(logits, axis=1, keepdims=True)
    idx1 = jnp.min(jnp.where(logits == m1, iota, NE + 1), axis=1,
                   keepdims=True)
    sel1 = iota == idx1
    masked = jnp.where(sel1, -jnp.inf, logits)
    m2 = jnp.max(masked, axis=1, keepdims=True)
    idx2 = jnp.min(jnp.where(masked == m2, iota, NE + 1), axis=1,
                   keepdims=True)
    sel2 = iota == idx2
    ex = jnp.exp(m2 - m1)
    denom = 1.0 + ex
    gates = (jnp.where(sel1, 1.0 / denom, 0.0)
             + jnp.where(sel2, ex / denom, 0.0))

    def cv_sq(v):
        m = jnp.mean(v)
        var = jnp.sum((v - m) ** 2) / np.float32(NE - 1)
        return var / (m * m + np.float32(1e-10))

    importance = jnp.sum(gates, axis=0)
    load = jnp.sum((gates > 0.0).astype(F32), axis=0)
    loss_ref[...] = jnp.full((1, 1),
                             (cv_sq(importance) + cv_sq(load)) * 0.01, F32)

    pm = PREC['moe']
    acc = jnp.zeros((B, D), F32)
    for e in range(NE):
        if e + 1 < NE:
            if e == 0:
                start(1, 1)
            else:
                guard = jnp.where(acc[0, 0] == np.float32(np.inf),
                                  1, 0).astype(jnp.int32)
                slot = (e + 1) % 2 + guard
                pltpu.make_async_copy(
                    w1s[e + 1], w1buf.at[slot],
                    sem1.at[(e + 1) % 2]).start()
                pltpu.make_async_copy(
                    w2s[e + 1], w2buf.at[slot],
                    sem2.at[(e + 1) % 2]).start()
        wait(e, e % 2)
        h = jnp.maximum(_mm(x, w1buf[e % 2], pm) + b1_ref[e:e + 1], 0.0)
        y = _mm(h, w2buf[e % 2], pm) + b2_ref[e:e + 1]
        acc = acc + gates[:, e:e + 1] * y
    o_ref[...] = acc


def _moe_call(x, p):
    in_size = D * GRAN
    b1 = jnp.stack([ep['fc1']['b'] for ep in p['experts']])
    b2 = jnp.stack([ep['fc2']['b'] for ep in p['experts']])
    any_spec = pl.BlockSpec(memory_space=pl.ANY)
    vmem_spec = pl.BlockSpec(memory_space=pltpu.MemorySpace.VMEM)
    y, loss = pl.pallas_call(
        _moe_kernel,
        in_specs=[vmem_spec, vmem_spec, vmem_spec, vmem_spec]
                 + [any_spec] * (2 * NE),
        out_shape=[jax.ShapeDtypeStruct((B, D), F32),
                   jax.ShapeDtypeStruct((1, 1), F32)],
        scratch_shapes=[pltpu.VMEM((2, in_size, D), F32),
                        pltpu.VMEM((2, D, D), F32),
                        pltpu.SemaphoreType.DMA((2,)),
                        pltpu.SemaphoreType.DMA((2,))],
    )(x, p['w_gate'], b1, b2,
      *[ep['fc1']['w'] for ep in p['experts']],
      *[ep['fc2']['w'] for ep in p['experts']])
    return y, loss


def _fusion_kernel(x_ref, wq_ref, wk_ref, wv_ref, wo_ref,
                   bq_ref, bk_ref, bv_ref, bo_ref,
                   g1_ref, b1_ref, w1_ref, f1_ref, w2_ref, f2_ref,
                   g2_ref, b2_ref, o_ref, *, SEQ):
    n = B * SEQ
    p = PREC['fusion']
    x = x_ref[...]
    q = _mm(x, wq_ref[...], p) + bq_ref[...]
    k = _mm(x, wk_ref[...], p) + bk_ref[...]
    v = _mm(x, wv_ref[...], p) + bv_ref[...]
    bi = jax.lax.broadcasted_iota(jnp.int32, (n, n), 0) // SEQ
    bj = jax.lax.broadcasted_iota(jnp.int32, (n, n), 1) // SEQ
    same = bi == bj
    outs = []
    for hh in range(NH):
        sl = slice(hh * DH, (hh + 1) * DH)
        s = _mm_t(q[:, sl], k[:, sl], p) * np.float32(DH ** -0.5)
        s = jnp.where(same, s, -jnp.inf)
        pr = jax.nn.softmax(s, axis=-1)
        outs.append(_mm(pr, v[:, sl], p))
    att = jnp.concatenate(outs, axis=1)
    x = _ln(x + _mm(att, wo_ref[...], p) + bo_ref[...], g1_ref[...], b1_ref[...])
    u = jnp.maximum(_mm(x, w1_ref[...], p) + f1_ref[...], 0.0)
    x = _ln(x + _mm(u, w2_ref[...], p) + f2_ref[...], g2_ref[...], b2_ref[...])
    o_ref[...] = x


def _fusion_layer(x, lp, SEQ, hidden):
    return pl.pallas_call(
        lambda *a: _fusion_kernel(*a, SEQ=SEQ),
        out_shape=jax.ShapeDtypeStruct((B * SEQ, D), F32),
    )(x, lp['q']['w'], lp['k']['w'], lp['v']['w'], lp['o']['w'],
      lp['q']['b'].reshape(1, D), lp['k']['b'].reshape(1, D),
      lp['v']['b'].reshape(1, D), lp['o']['b'].reshape(1, D),
      lp['ln1g'].reshape(1, D), lp['ln1b'].reshape(1, D),
      lp['ff1']['w'], lp['ff1']['b'].reshape(1, hidden),
      lp['ff2']['w'], lp['ff2']['b'].reshape(1, D),
      lp['ln2g'].reshape(1, D), lp['ln2b'].reshape(1, D))


def _head_kernel(h0_ref, h1_ref, h2_ref, h3_ref, h4_ref, h5_ref,
                 mcs_w_ref, mcs_b_ref, fcs_w_ref, fcs_b_ref,
                 mfs_w_ref, mfs_b_ref, ffs_w_ref, ffs_b_ref,
                 all_w_ref, all_b_ref, ts_w_ref, ts_b_ref,
                 as_w_ref, as_b_ref, vs_w_ref, vs_b_ref,
                 cls_w_ref, cls_b_ref, o_ref):
    h0, h1, h2 = h0_ref[...], h1_ref[...], h2_ref[...]
    h3, h4, h5 = h3_ref[...], h4_ref[...], h5_ref[...]
    scale = np.float32(D ** -0.5)

    ph = PREC['head']

    def attn_mod(f1, f2, f3, mw, mb, fw, fb):
        a = jnp.maximum(
            _mm(jnp.concatenate([f1, f2, f3], axis=1), mw, ph) + mb, 0.0)
        s = (_mm(a, fw, ph) + fb) * scale
        w = jax.nn.softmax(s, axis=1)
        fused = (f1 * w[:, 0:1] + f2 * w[:, 1:2] + f3 * w[:, 2:3])
        return w, fused

    aw_cs, a_cs = attn_mod(h0, h2, h4, mcs_w_ref[...], mcs_b_ref[...],
                           fcs_w_ref[...], fcs_b_ref[...])
    _, a_fs = attn_mod(h1, h3, h5, mfs_w_ref[...], mfs_b_ref[...],
                       ffs_w_ref[...], ffs_b_ref[...])
    fea_cfs = _mm(jnp.concatenate([a_cs, a_fs], axis=1),
                  all_w_ref[...], ph) + all_b_ref[...]
    fm_t = _mm(jnp.concatenate([h0, h1], axis=1), ts_w_ref[...], ph) + ts_b_ref[...]
    fm_a = _mm(jnp.concatenate([h2, h3], axis=1), as_w_ref[...], ph) + as_b_ref[...]
    fm_v = _mm(jnp.concatenate([h4, h5], axis=1), vs_w_ref[...], ph) + vs_b_ref[...]
    fcf = (fm_t * aw_cs[:, 0:1] + fm_a * aw_cs[:, 1:2] + fm_v * aw_cs[:, 2:3])
    o_ref[...] = _mm(jnp.concatenate([fea_cfs, fcf], axis=1),
                     cls_w_ref[...], ph) + cls_b_ref[...]


def _head(hs, params):
    p = params
    return pl.pallas_call(
        _head_kernel,
        out_shape=jax.ShapeDtypeStruct((B, NCLS), F32),
    )(*hs,
      p['attn_cs']['mlp']['w'], p['attn_cs']['mlp']['b'].reshape(1, D),
      p['attn_cs']['fc']['w'], p['attn_cs']['fc']['b'].reshape(1, 3),
      p['attn_fs']['mlp']['w'], p['attn_fs']['mlp']['b'].reshape(1, D),
      p['attn_fs']['fc']['w'], p['attn_fs']['fc']['b'].reshape(1, 3),
      p['fc_all']['w'], p['fc_all']['b'].reshape(1, D),
      p['fc_ts']['w'], p['fc_ts']['b'].reshape(1, D),
      p['fc_as']['w'], p['fc_as']['b'].reshape(1, D),
      p['fc_vs']['w'], p['fc_vs']['b'].reshape(1, D),
      p['classifier']['w'], p['classifier']['b'].reshape(1, NCLS))


def _encoder(x, p, S, Cin, tile):
    h = _conv_pe(x, p['conv'], S, Cin)
    for lp in p['enc']['layers']:
        h = _attn_preln(h, lp, S)
        h = _ffn_preln(h, lp, S, 4 * D, tile)
    co, fi = _lnpool(h, p['enc']['lnfg'], p['enc']['lnfb'], S)
    return co, fi


def _moe(fine, p):
    x = fine.transpose(0, 2, 1).reshape(B, D * GRAN)
    return _moe_call(x, p)


def kernel(x_t, x_a, x_v, label_t, label_a, label_v, label_m, params):
    x_t = x_t[:, 0:80, :].astype(F32)
    x_a = x_a.astype(F32)
    x_v = x_v.astype(F32)

    co_a, fi_a = _encoder(x_a, params['audio'], 256, 1024, 256)
    co_v, fi_v = _encoder(x_v, params['video'], 256, 2048, 256)
    co_t, fi_t = _encoder(x_t, params['text'], 80, 1024, 256)

    fine_ts, l_ts = _moe(fi_t, params['moe_t'])
    fine_as, l_as = _moe(fi_a, params['moe_a'])
    fine_vs, l_vs = _moe(fi_v, params['moe_v'])

    h = jnp.stack([co_t, fine_ts, co_a, fine_as, co_v, fine_vs],
                  axis=1).reshape(B * 6, D)
    for lp in params['trans']:
        h = _fusion_layer(h, lp, 6, 2048)
    h = h.reshape(B, 6, D)
    hs = [h[:, i, :] for i in range(6)]

    logits = _head(hs, params)
    moes = (l_ts + l_as + l_vs).reshape(())
    return (moes, logits)
